# baseline (speedup 1.0000x reference)
import functools

import jax
import jax.numpy as jnp
from jax import lax
from jax.experimental import pallas as pl
from jax.experimental.pallas import tpu as pltpu
from jax.experimental.pallas import tpu_sc as plsc

NC = 2
NS = 16
NW = NC * NS
LANES = 16
CH = 128
PB = 40
NBLK = 62
PBT = 20
CPT = 4
CPK = CPT // 2


def kernel(X, edge_index, edge_weight, W, b):
    N, D = X.shape
    E = edge_index.shape[1]
    DO = W.shape[1]
    bits = (N - 1).bit_length()
    sd_r = (edge_index[0] | (edge_index[1] << bits)).reshape(E // CH, CH)
    xb = X.astype(jnp.bfloat16).reshape(N, D // 2, 2)
    XTP = jnp.transpose(lax.bitcast_convert_type(xb, jnp.int32))
    fT = _sc_spmm_cols(XTP, sd_r, edge_weight, N=N, D=D, bits=bits)
    out = _tc_linear(fT, W, b.reshape(1, DO), N=N, D=D, DO=DO)
    return out


def _sc_spmm_cols(XTP, sd_r, w_p, *, N, D, bits):
    mesh = plsc.VectorSubcoreMesh(core_axis_name="c", subcore_axis_name="s")
    lomask = jnp.int32((1 << bits) - 1)
    himask = jnp.int32(-65536)

    @functools.partial(
        pl.kernel,
        out_type=jax.ShapeDtypeStruct((D, N), jnp.float32),
        mesh=mesh,
        compiler_params=pltpu.CompilerParams(use_tc_tiling_on_sc=False,
                                             needs_layout_passes=False),
        scratch_types=[
            pltpu.VMEM((CPK, N), jnp.int32),
            pltpu.VMEM((CPT, N), jnp.float32),
            pltpu.VMEM((PB, CH), jnp.int32),
            pltpu.VMEM((PB * CH,), jnp.float32),
            pltpu.VMEM((PB, CH), jnp.int32),
            pltpu.VMEM((PB * CH,), jnp.float32),
            pltpu.SemaphoreType.DMA,
            pltpu.SemaphoreType.DMA,
        ],
    )
    def spmm(xtp_hbm, sd_hbm, w_hbm, ft_hbm,
             xt2, ft4, sdA, wA, sdB, wB, semA, semB):
        c = lax.axis_index("c")
        s = lax.axis_index("s")
        wid = s * NC + c

        pltpu.sync_copy(xtp_hbm.at[pl.ds(wid * CPK, CPK)], xt2)
        zero = jnp.zeros((LANES,), jnp.float32)

        def zero_body(i, carry):
            for col in range(CPT):
                ft4[col, pl.ds(i * LANES, LANES)] = zero
            return carry

        lax.fori_loop(0, N // LANES, zero_body, 0)

        def stage(blk, sdbuf, wbuf, sem):
            pltpu.async_copy(sd_hbm.at[pl.ds(blk * PB, PB)], sdbuf, sem)
            pltpu.async_copy(w_hbm.at[pl.ds(blk * PB * CH, PB * CH)],
                             wbuf, sem)

        def stage_wait(sdbuf, wbuf, sem):
            pltpu.make_async_copy(sd_hbm.at[pl.ds(0, PB)], sdbuf, sem).wait()
            pltpu.make_async_copy(w_hbm.at[pl.ds(0, PB * CH)],
                                  wbuf, sem).wait()

        def process(sdbuf, wbuf, nrows=PB):
            @plsc.parallel_loop(0, nrows, 1, unroll=8)
            def _row(r):
                for g in range(CH // LANES):
                    sd16 = sdbuf[r, pl.ds(g * LANES, LANES)]
                    src16 = sd16 & lomask
                    dst16 = lax.shift_right_logical(sd16, bits)
                    w16 = wbuf[pl.ds(r * CH + g * LANES, LANES)]
                    for h in range(CPK):
                        rowi = jnp.full((LANES,), h, jnp.int32)
                        xi = plsc.load_gather(xt2, [rowi, src16])
                        xa = plsc.bitcast(xi << 16, jnp.float32)
                        xbv = plsc.bitcast(xi & himask, jnp.float32)
                        ra = jnp.full((LANES,), 2 * h, jnp.int32)
                        rb = jnp.full((LANES,), 2 * h + 1, jnp.int32)
                        plsc.addupdate_scatter(ft4, [ra, dst16], xa * w16)
                        plsc.addupdate_scatter(ft4, [rb, dst16], xbv * w16)

        stage(0, sdA, wA, semA)
        stage(1, sdB, wB, semB)

        def blk_body(q, carry):
            blk = q * 2
            stage_wait(sdA, wA, semA)
            process(sdA, wA)

            @pl.when(q < NBLK // 2 - 1)
            def _():
                stage(blk + 2, sdA, wA, semA)

            stage_wait(sdB, wB, semB)
            process(sdB, wB)

            @pl.when(q < NBLK // 2 - 1)
            def _():
                stage(blk + 3, sdB, wB, semB)
            return carry

        lax.fori_loop(0, NBLK // 2, blk_body, 0)

        pltpu.sync_copy(sd_hbm.at[pl.ds(NBLK * PB, PBT)],
                        sdA.at[pl.ds(0, PBT)])
        pltpu.sync_copy(w_hbm.at[pl.ds(NBLK * PB * CH, PBT * CH)],
                        wA.at[pl.ds(0, PBT * CH)])
        process(sdA, wA, nrows=PBT)

        pltpu.sync_copy(ft4, ft_hbm.at[pl.ds(wid * CPT, CPT)])

    return spmm(XTP, sd_r, w_p)


def _tc_linear(fT, W, b2, *, N, D, DO):
    def body(f_ref, w_ref, b_ref, o_ref):
        o_ref[...] = lax.dot_general(
            f_ref[...], w_ref[...],
            dimension_numbers=(((0,), (0,)), ((), ())),
            preferred_element_type=jnp.float32) + b_ref[...]

    return pl.pallas_call(
        body,
        in_specs=[
            pl.BlockSpec((D, N), lambda: (0, 0)),
            pl.BlockSpec((D, DO), lambda: (0, 0)),
            pl.BlockSpec((1, DO), lambda: (0, 0)),
        ],
        out_specs=pl.BlockSpec((N, DO), lambda: (0, 0)),
        out_shape=jax.ShapeDtypeStruct((N, DO), jnp.float32),
    )(fT, W, b2)


# speedup vs baseline: 1.0214x; 1.0214x over previous
import functools

import jax
import jax.numpy as jnp
from jax import lax
from jax.experimental import pallas as pl
from jax.experimental.pallas import tpu as pltpu
from jax.experimental.pallas import tpu_sc as plsc

NC = 2
NS = 16
NW = NC * NS
LANES = 16
CH = 128
PB = 40
NBLK = 62
PBT = 20
CPT = 4
CPK = CPT // 2


def kernel(X, edge_index, edge_weight, W, b):
    N, D = X.shape
    E = edge_index.shape[1]
    DO = W.shape[1]
    bits = (N - 1).bit_length()
    sd_r = (edge_index[0] | (edge_index[1] << bits)).reshape(E // CH, CH)
    xb = X.astype(jnp.bfloat16).reshape(N, D // 2, 2)
    XTP = jnp.transpose(lax.bitcast_convert_type(xb, jnp.int32))
    fT = _sc_spmm_cols(XTP, sd_r, edge_weight, N=N, D=D, bits=bits)
    out = _tc_linear(fT, W, b.reshape(1, DO), N=N, D=D, DO=DO)
    return out


def _sc_spmm_cols(XTP, sd_r, w_p, *, N, D, bits):
    mesh = plsc.VectorSubcoreMesh(core_axis_name="c", subcore_axis_name="s")
    lomask = jnp.int32((1 << bits) - 1)
    himask = jnp.int32(-65536)

    @functools.partial(
        pl.kernel,
        out_type=jax.ShapeDtypeStruct((D, N), jnp.float32),
        mesh=mesh,
        compiler_params=pltpu.CompilerParams(use_tc_tiling_on_sc=False,
                                             needs_layout_passes=False),
        scratch_types=[
            pltpu.VMEM((CPK, N), jnp.int32),
            pltpu.VMEM((CPT, N), jnp.float32),
            pltpu.VMEM((PB, CH), jnp.int32),
            pltpu.VMEM((PB * CH,), jnp.float32),
            pltpu.VMEM((PB, CH), jnp.int32),
            pltpu.VMEM((PB * CH,), jnp.float32),
            pltpu.SemaphoreType.DMA,
            pltpu.SemaphoreType.DMA,
        ],
    )
    def spmm(xtp_hbm, sd_hbm, w_hbm, ft_hbm,
             xt2, ft4, sdA, wA, sdB, wB, semA, semB):
        c = lax.axis_index("c")
        s = lax.axis_index("s")
        wid = s * NC + c

        pltpu.sync_copy(xtp_hbm.at[pl.ds(wid * CPK, CPK)], xt2)
        zero = jnp.zeros((LANES,), jnp.float32)

        def zero_body(i, carry):
            for col in range(CPT):
                ft4[col, pl.ds(i * LANES, LANES)] = zero
            return carry

        lax.fori_loop(0, N // LANES, zero_body, 0)

        def stage(blk, sdbuf, wbuf, sem):
            pltpu.async_copy(sd_hbm.at[pl.ds(blk * PB, PB)], sdbuf, sem)
            pltpu.async_copy(w_hbm.at[pl.ds(blk * PB * CH, PB * CH)],
                             wbuf, sem)

        def stage_wait(sdbuf, wbuf, sem):
            pltpu.make_async_copy(sd_hbm.at[pl.ds(0, PB)], sdbuf, sem).wait()
            pltpu.make_async_copy(w_hbm.at[pl.ds(0, PB * CH)],
                                  wbuf, sem).wait()

        def process(sdbuf, wbuf, nrows=PB):
            @plsc.parallel_loop(0, nrows, 1, unroll=4)
            def _row(r):
                for g in range(CH // LANES):
                    sd16 = sdbuf[r, pl.ds(g * LANES, LANES)]
                    src16 = sd16 & lomask
                    dst16 = lax.shift_right_logical(sd16, bits)
                    w16 = wbuf[pl.ds(r * CH + g * LANES, LANES)]
                    for h in range(CPK):
                        rowi = jnp.full((LANES,), h, jnp.int32)
                        xi = plsc.load_gather(xt2, [rowi, src16])
                        xa = plsc.bitcast(xi << 16, jnp.float32)
                        xbv = plsc.bitcast(xi & himask, jnp.float32)
                        ra = jnp.full((LANES,), 2 * h, jnp.int32)
                        rb = jnp.full((LANES,), 2 * h + 1, jnp.int32)
                        plsc.addupdate_scatter(ft4, [ra, dst16], xa * w16)
                        plsc.addupdate_scatter(ft4, [rb, dst16], xbv * w16)

        stage(0, sdA, wA, semA)
        stage(1, sdB, wB, semB)

        def blk_body(q, carry):
            blk = q * 2
            stage_wait(sdA, wA, semA)
            process(sdA, wA)

            @pl.when(q < NBLK // 2 - 1)
            def _():
                stage(blk + 2, sdA, wA, semA)

            stage_wait(sdB, wB, semB)
            process(sdB, wB)

            @pl.when(q < NBLK // 2 - 1)
            def _():
                stage(blk + 3, sdB, wB, semB)
            return carry

        lax.fori_loop(0, NBLK // 2, blk_body, 0)

        pltpu.sync_copy(sd_hbm.at[pl.ds(NBLK * PB, PBT)],
                        sdA.at[pl.ds(0, PBT)])
        pltpu.sync_copy(w_hbm.at[pl.ds(NBLK * PB * CH, PBT * CH)],
                        wA.at[pl.ds(0, PBT * CH)])
        process(sdA, wA, nrows=PBT)

        pltpu.sync_copy(ft4, ft_hbm.at[pl.ds(wid * CPT, CPT)])

    return spmm(XTP, sd_r, w_p)


def _tc_linear(fT, W, b2, *, N, D, DO):
    def body(f_ref, w_ref, b_ref, o_ref):
        o_ref[...] = lax.dot_general(
            f_ref[...], w_ref[...],
            dimension_numbers=(((0,), (0,)), ((), ())),
            preferred_element_type=jnp.float32) + b_ref[...]

    return pl.pallas_call(
        body,
        in_specs=[
            pl.BlockSpec((D, N), lambda: (0, 0)),
            pl.BlockSpec((D, DO), lambda: (0, 0)),
            pl.BlockSpec((1, DO), lambda: (0, 0)),
        ],
        out_specs=pl.BlockSpec((N, DO), lambda: (0, 0)),
        out_shape=jax.ShapeDtypeStruct((N, DO), jnp.float32),
    )(fT, W, b2)
